# baseline (speedup 1.0000x reference)
import functools

import jax
import jax.numpy as jnp
from jax.experimental import pallas as pl
from jax.experimental.pallas import tpu as pltpu

R = 128
N = 100000
BC = 4096
NBF = 24
BULK = NBF * BC
TW = 2048
TAIL = N - BULK
K = 8
BF = 512
_I32_MAX = jnp.iinfo(jnp.int32).max


def _in_copy(x_hbm, ibuf, in_sems, j, slot):
    return pltpu.make_async_copy(
        x_hbm.at[:, pl.ds(j * BC, BC)], ibuf.at[slot], in_sems.at[slot])


def _out_copy(o_hbm, zbuf, out_sems, j):
    return pltpu.make_async_copy(
        zbuf, o_hbm.at[:, pl.ds(j * BC, BC)], out_sems.at[j])


def _scan_kernel(x_hbm, xt_ref, o_hbm, idx_ref, zbuf, ibuf, acc_ref,
                 blk_ref, in_sems, out_sems):
    zbuf[...] = jnp.zeros_like(zbuf)
    acc_ref[...] = jnp.full_like(acc_ref, -jnp.inf)
    blk_ref[...] = jnp.zeros_like(blk_ref)

    for s in range(K):
        _in_copy(x_hbm, ibuf, in_sems, s, s).start()

    def body(j, _):
        slot = jax.lax.rem(j, K)
        _in_copy(x_hbm, ibuf, in_sems, j, slot).wait()
        x = ibuf[slot]
        better = x > acc_ref[...]
        acc_ref[...] = jnp.where(better, x, acc_ref[...])
        blk_ref[...] = jnp.where(better, j, blk_ref[...])
        _out_copy(o_hbm, zbuf, out_sems, j).start()
        nxt = j + K

        @pl.when(nxt < NBF)
        def _():
            _in_copy(x_hbm, ibuf, in_sems, nxt, slot).start()

        return 0

    jax.lax.fori_loop(0, NBF, body, 0)

    acc = acc_ref[...]
    lane = jax.lax.broadcasted_iota(jnp.int32, (R, BC), 1)
    m = jnp.max(acc, axis=-1, keepdims=True)
    cand = jnp.where(acc == m, blk_ref[...] * BC + lane, _I32_MAX)
    idx_b = jnp.min(cand, axis=-1, keepdims=True)

    lane_t = jax.lax.broadcasted_iota(jnp.int32, (R, TW), 1)
    xt = jnp.where(lane_t < TAIL, xt_ref[...], -jnp.inf)
    mt = jnp.max(xt, axis=-1, keepdims=True)
    cand_t = jnp.where(xt == mt, lane_t + BULK, _I32_MAX)
    idx_t = jnp.min(cand_t, axis=-1, keepdims=True)
    take_t = mt > m
    idx_ref[...] = jnp.where(take_t, idx_t, idx_b)

    for j in range(NBF):
        _out_copy(o_hbm, zbuf, out_sems, j).wait()


def _tail_kernel(z_hbm, idxv_ref, o_ref):
    del z_hbm
    lane = jax.lax.broadcasted_iota(jnp.int32, (R, TW), 1)
    o_ref[...] = (lane + BULK == idxv_ref[...]).astype(jnp.float32)


def _fix_kernel(sref, z_hbm, o_ref):
    del z_hbm
    r = pl.program_id(0)
    c = sref[r]
    base = (jnp.minimum(c, BULK - 1) // BF) * BF
    lane = jax.lax.broadcasted_iota(jnp.int32, (1, 1, BF), 2)
    o_ref[...] = (lane + base == c).astype(jnp.float32)


@functools.partial(jax.jit, static_argnames=("interpret",))
def kernel(logits, interpret=False):
    zeros_out, idx = pl.pallas_call(
        _scan_kernel,
        grid=(1,),
        in_specs=[pl.BlockSpec(memory_space=pl.ANY),
                  pl.BlockSpec((R, TW), lambda i: (0, BULK // TW))],
        out_specs=[pl.BlockSpec(memory_space=pl.ANY),
                   pl.BlockSpec((R, 1), lambda i: (0, 0))],
        out_shape=[jax.ShapeDtypeStruct((R, N), jnp.float32),
                   jax.ShapeDtypeStruct((R, 1), jnp.int32)],
        scratch_shapes=[
            pltpu.VMEM((R, BC), jnp.float32),
            pltpu.VMEM((K, R, BC), jnp.float32),
            pltpu.VMEM((R, BC), jnp.float32),
            pltpu.VMEM((R, BC), jnp.int32),
            pltpu.SemaphoreType.DMA((K,)),
            pltpu.SemaphoreType.DMA((NBF,)),
        ],
        interpret=interpret,
    )(logits, logits)

    with_tail = pl.pallas_call(
        _tail_kernel,
        grid=(1,),
        in_specs=[pl.BlockSpec(memory_space=pl.ANY),
                  pl.BlockSpec((R, 1), lambda i: (0, 0))],
        out_specs=pl.BlockSpec((R, TW), lambda i: (0, BULK // TW)),
        out_shape=jax.ShapeDtypeStruct((R, N), jnp.float32),
        input_output_aliases={0: 0},
        interpret=interpret,
    )(zeros_out, idx)

    grid_spec = pltpu.PrefetchScalarGridSpec(
        num_scalar_prefetch=1,
        grid=(R,),
        in_specs=[pl.BlockSpec(memory_space=pl.ANY)],
        out_specs=pl.BlockSpec(
            (1, 1, BF),
            lambda r, sref: (r, 0, jnp.minimum(sref[r], BULK - 1) // BF)),
    )
    out3 = pl.pallas_call(
        _fix_kernel,
        grid_spec=grid_spec,
        out_shape=jax.ShapeDtypeStruct((R, 1, N), jnp.float32),
        input_output_aliases={1: 0},
        interpret=interpret,
    )(idx.reshape(R), with_tail.reshape(R, 1, N))
    return out3.reshape(R, N)
